# baseline (speedup 1.0000x reference)
import functools

import jax
import jax.numpy as jnp
from jax import lax
from jax.experimental import pallas as pl
from jax.experimental.pallas import tpu as pltpu
from jax.experimental.pallas import tpu_sc as plsc

MAX_DISTANCE = 128
NUM_HEADS = 12
EMBEDDING_DIM = 32
SEQ_LEN = 512

_ROWS = 2 * MAX_DISTANCE + 1
_D = NUM_HEADS * EMBEDDING_DIM
_ROWS_PER_W = 16
_CHUNK_J = 128
_WIN = _ROWS_PER_W + _CHUNK_J


def _make_sc_call():
    info = plsc.get_sparse_core_info()
    nc, ns = info.num_cores, info.num_subcores
    mesh = plsc.VectorSubcoreMesh(core_axis_name="c", subcore_axis_name="s")

    @functools.partial(
        pl.kernel,
        mesh=mesh,
        compiler_params=pltpu.CompilerParams(use_tc_tiling_on_sc=False),
        out_type=jax.ShapeDtypeStruct(
            (SEQ_LEN, SEQ_LEN, NUM_HEADS, EMBEDDING_DIM), jnp.float32
        ),
        scratch_types=[
            pltpu.VMEM((_CHUNK_J,), jnp.int32),
            pltpu.VMEM((16,), jnp.int32),
            pltpu.VMEM((2, _WIN, NUM_HEADS, EMBEDDING_DIM), jnp.float32),
            pltpu.SemaphoreType.DMA,
            pltpu.SemaphoreType.DMA,
        ],
    )
    def call(table, out, idxa, idxb, buf, gsem, wsem):
        wid = lax.axis_index("s") * nc + lax.axis_index("c")
        i0 = wid * _ROWS_PER_W
        iota = lax.iota(jnp.int32, 16)

        def fire_gather(c):
            base = 143 + i0 - c * _CHUNK_J
            for s in range(8):
                idxa[pl.ds(s * 16, 16)] = jnp.clip(base - s * 16 - iota, 0, _ROWS - 1)
            idxb[...] = jnp.clip(base - 128 - iota, 0, _ROWS - 1)
            dst = buf.at[c % 2]
            return (
                pltpu.async_copy(table.at[idxa], dst.at[pl.ds(0, _CHUNK_J)], gsem),
                pltpu.async_copy(table.at[idxb], dst.at[pl.ds(_CHUNK_J, 16)], gsem),
            )

        gs = fire_gather(0)
        for c in range(4):
            gs[0].wait()
            gs[1].wait()
            src = buf.at[c % 2]
            writes = [
                pltpu.async_copy(
                    src.at[pl.ds(_ROWS_PER_W - 1 - r, _CHUNK_J)],
                    out.at[i0 + r, pl.ds(c * _CHUNK_J, _CHUNK_J)],
                    wsem,
                )
                for r in range(_ROWS_PER_W)
            ]
            if c < 3:
                gs = fire_gather(c + 1)
            for w in writes:
                w.wait()

    return call


def kernel(rel_embeddings, seq_len):
    del seq_len
    return _make_sc_call()(rel_embeddings)


# speedup vs baseline: 3.4945x; 3.4945x over previous
import functools

import jax
import jax.numpy as jnp
from jax import lax
from jax.experimental import pallas as pl
from jax.experimental.pallas import tpu as pltpu
from jax.experimental.pallas import tpu_sc as plsc

MAX_DISTANCE = 128
NUM_HEADS = 12
EMBEDDING_DIM = 32
SEQ_LEN = 512

_ROWS = 2 * MAX_DISTANCE + 1
_D = NUM_HEADS * EMBEDDING_DIM
_Q_COLS = 2 * SEQ_LEN
_SLAB = _Q_COLS // 16
_SHIFTS_PER_CORE = 4
_TBAND = 72
_HD_BLK = 48


def _make_sc_call():
    info = plsc.get_sparse_core_info()
    nc, ns = info.num_cores, info.num_subcores
    mesh = plsc.VectorSubcoreMesh(core_axis_name="c", subcore_axis_name="s")

    @functools.partial(
        pl.kernel,
        mesh=mesh,
        compiler_params=pltpu.CompilerParams(
            use_tc_tiling_on_sc=False, needs_layout_passes=False
        ),
        out_type=jax.ShapeDtypeStruct((SEQ_LEN, _D, SEQ_LEN), jnp.float32),
        scratch_types=[
            pltpu.VMEM((_TBAND, NUM_HEADS, EMBEDDING_DIM), jnp.float32),
            pltpu.VMEM((_HD_BLK, _SLAB), jnp.float32),
            pltpu.VMEM_SHARED((_SHIFTS_PER_CORE, _D, _Q_COLS), jnp.float32),
            pltpu.SemaphoreType.DMA,
        ],
    )
    def call(table, out, tbuf, qbuf, q_sh, wsem):
        cid = lax.axis_index("c")
        sid = lax.axis_index("s")
        iota = lax.iota(jnp.int32, 16)

        k0 = sid * _SLAB
        lo = jnp.clip(569 - k0, 0, _ROWS - _TBAND)
        pltpu.sync_copy(table.at[pl.ds(lo, _TBAND)], tbuf)
        for u in range(_SHIFTS_PER_CORE):
            base = 639 - (_SHIFTS_PER_CORE * cid + u) - k0
            rowvs = [
                jnp.clip(base - (16 * v + iota), 0, _ROWS - 1) - lo
                for v in range(_SLAB // 16)
            ]
            for blk in range(_D // _HD_BLK):

                def build(hdl, acc, rowvs=rowvs, blk=blk):
                    hd = blk * _HD_BLK + hdl
                    hv = jnp.broadcast_to(hd // EMBEDDING_DIM, (16,))
                    dv = jnp.broadcast_to(hd % EMBEDDING_DIM, (16,))
                    for v, rowv in enumerate(rowvs):
                        qbuf[hdl, pl.ds(16 * v, 16)] = plsc.load_gather(
                            tbuf, [rowv, hv, dv]
                        )
                    return acc

                lax.fori_loop(0, _HD_BLK, build, 0)
                pltpu.sync_copy(
                    qbuf,
                    q_sh.at[u, pl.ds(blk * _HD_BLK, _HD_BLK), pl.ds(k0, _SLAB)],
                )
        plsc.subcore_barrier()

        writes = []
        for t1 in range(4):
            g = 4 * sid + t1
            s_off = pl.multiple_of(504 - 8 * g, 8)
            for t2 in range(4):
                i = 8 * g + t2 + 4 * (1 - cid)
                writes.append(
                    pltpu.async_copy(
                        q_sh.at[3 - t2, :, pl.ds(s_off, SEQ_LEN)],
                        out.at[i],
                        wsem,
                    )
                )
        for w in writes:
            w.wait()

    return call


def kernel(rel_embeddings, seq_len):
    del seq_len
    out_t = _make_sc_call()(rel_embeddings)
    out_t = out_t.reshape(SEQ_LEN, NUM_HEADS, EMBEDDING_DIM, SEQ_LEN)
    return out_t.transpose(0, 3, 1, 2)
